# baseline (speedup 1.0000x reference)
import math

import jax
import jax.numpy as jnp
from jax.experimental import pallas as pl

F_MAX = 50
VOCAB = 73
IN_DIM = 14
HID = 128
LAT = 256
DM = 256
NHEAD = 4
DH = DM // NHEAD
NLAYERS = 2
CB = 512
NQ = 4
FF = DM * 4
COMMIT_W = 0.25

FPAD = 64
BK1 = 128
BK2 = 4
R2 = BK2 * F_MAX

_BF = jnp.bfloat16

_ENC_KEYS = ('proj_in_w', 'proj_in_b', 'norm0_g', 'norm0_b',
             'l1_w', 'l1_b', 'norm1_g', 'norm1_b',
             'l2_w', 'l2_b', 'norm2_g', 'norm2_b',
             'attn_q', 'out_w', 'out_b')

_LAYER_KEYS = ('ln1_g', 'ln1_b',
               'sa_wq', 'sa_bq', 'sa_wk', 'sa_bk', 'sa_wv', 'sa_bv',
               'sa_wo', 'sa_bo',
               'ln2_g', 'ln2_b',
               'ca_wq', 'ca_bq', 'ca_wk', 'ca_bk', 'ca_wv', 'ca_bv',
               'ca_wo', 'ca_bo',
               'ln3_g', 'ln3_b',
               'ff1_w', 'ff1_b', 'ff2_w', 'ff2_b')


def _bdot(a, b):
    return jnp.dot(a.astype(_BF), b.astype(_BF),
                   preferred_element_type=jnp.float32)


def _ln(x, g, b, eps=1e-5):
    m = jnp.mean(x, axis=-1, keepdims=True)
    v = jnp.mean((x - m) ** 2, axis=-1, keepdims=True)
    return (x - m) * jax.lax.rsqrt(v + eps) * g + b


def _gelu(x):
    return 0.5 * x * (1.0 + jax.lax.erf(x * (2.0 ** -0.5)))


def _enc_rvq_body(*refs):
    (f_ref, phid_ref, len_ref, phemb_ref) = refs[0:4]
    enc_s = refs[4:19]
    enc_e = refs[19:34]
    (cbs_ref, cbts_ref, cbe_ref, cbte_ref,
     p2dw_ref, p2db_ref, memw_ref, memb_ref, mtype_ref) = refs[34:43]
    (mems_o, meme_o, memp_o, sidx_o, eidx_o, commit_o) = refs[43:49]

    bk = phid_ref.shape[0]
    f2 = f_ref[...]

    onehot = (phid_ref[...] == jax.lax.broadcasted_iota(
        jnp.int32, (bk, VOCAB), 1)).astype(jnp.float32)
    ph_e = jnp.dot(onehot, phemb_ref[...],
                   preferred_element_type=jnp.float32)

    lens = len_ref[...]
    fid = jax.lax.broadcasted_iota(jnp.int32, (bk, FPAD), 1)
    validm = fid < lens
    mid = jnp.maximum(lens // 2, 1)
    smask = (fid < mid) & validm
    one1 = lens == 1
    emask = (one1 & smask) | ((~one1) & (fid >= mid) & validm)

    def half(e, mask):
        (piw, pib, n0g, n0b, l1w, l1b, n1g, n1b,
         l2w, l2b, n2g, n2b, aq8, ow, ob) = (r[...] for r in e)
        x = _bdot(f2, piw) + pib
        x = (x.reshape(bk, FPAD, HID) + ph_e[:, None, :]).reshape(bk * FPAD, HID)
        x = _ln(x, n0g, n0b)
        x = _ln(x + _gelu(_bdot(x, l1w) + l1b), n1g, n1b)
        x = _ln(x + _gelu(_bdot(x, l2w) + l2b), n2g, n2b)
        sc8 = _bdot(x, aq8).reshape(bk, FPAD, 8)
        pick = (jax.lax.broadcasted_iota(jnp.int32, (1, FPAD, 8), 1) % 8
                == jax.lax.broadcasted_iota(jnp.int32, (1, FPAD, 8), 2))
        s = jnp.sum(jnp.where(pick, sc8, 0.0), axis=2) * (HID ** -0.5)
        s = jnp.where(mask, s, -1e9)
        s = s - jnp.max(s, axis=1, keepdims=True)
        ew = jnp.exp(s)
        w = ew / jnp.sum(ew, axis=1, keepdims=True)
        x3t = jnp.transpose(x.reshape(bk, FPAD, HID), (0, 2, 1))
        pooled = jnp.sum(x3t * w[:, None, :], axis=2)
        return _bdot(pooled, ow) + ob

    z_s = half(enc_s, smask)
    z_e = half(enc_e, emask)

    lane = jax.lax.broadcasted_iota(jnp.int32, (bk, CB), 1)

    def rvq(z, cb_ref, cbt_ref, idx_o):
        r = z
        ssq = jnp.zeros((1, 1), jnp.float32)
        cols = []
        for qi in range(NQ):
            cbt = cbt_ref[:, qi * CB:(qi + 1) * CB]
            cn = jnp.sum(cbt * cbt, axis=0, keepdims=True)
            d = cn - 2.0 * _bdot(r, cbt)
            dmin = jnp.min(d, axis=1, keepdims=True)
            idx = jnp.min(jnp.where(d <= dmin, lane, CB), axis=1,
                          keepdims=True)
            oh = (lane == idx).astype(jnp.float32)
            quant = jnp.dot(oh, cb_ref[qi * CB:(qi + 1) * CB, :],
                            preferred_element_type=jnp.float32)
            ssq = ssq + jnp.sum((quant - r) ** 2).reshape(1, 1)
            r = r - quant
            cols.append(idx)
        idx_o[...] = jnp.concatenate(cols, axis=1)
        return z - r, ssq

    q_s, ssq_s = rvq(z_s, cbs_ref, cbts_ref, sidx_o)
    q_e, ssq_e = rvq(z_e, cbe_ref, cbte_ref, eidx_o)

    memw = memw_ref[...]
    memb = memb_ref[...]
    mt = mtype_ref[...]
    mems_o[...] = _bdot(q_s, memw) + memb + mt[0:1, :]
    meme_o[...] = _bdot(q_e, memw) + memb + mt[1:2, :]
    memp_o[...] = _bdot(ph_e, p2dw_ref[...]) + p2db_ref[...] + mt[2:3, :]

    total = ssq_s + ssq_e

    @pl.when(pl.program_id(0) == 0)
    def _init():
        commit_o[...] = jnp.zeros_like(commit_o)

    commit_o[...] += total


def _dec_body(*refs):
    (mem_ref, len_ref, ap1w_ref, ap1b_ref, ap2w_ref, ap2b_ref) = refs[0:6]
    lyr = [refs[6 + i * 26: 6 + (i + 1) * 26] for i in range(NLAYERS)]
    outw_ref, outb_ref = refs[6 + NLAYERS * 26: 8 + NLAYERS * 26]
    recon_o = refs[8 + NLAYERS * 26]

    mem_cat = mem_ref[0]
    lens_row = len_ref[0]

    rsub = jax.lax.broadcasted_iota(jnp.int32, (R2, 1), 0)
    b_loc = rsub // F_MAX
    f_loc = rsub % F_MAX
    bcols = jax.lax.broadcasted_iota(jnp.int32, (R2, BK2), 1)
    E = (b_loc == bcols).astype(jnp.float32)

    lenr = jnp.sum(jnp.where(b_loc == bcols,
                             lens_row.astype(jnp.float32), 0.0),
                   axis=1, keepdims=True)
    den = jnp.maximum(lenr - 1.0, 1.0)
    alpha = jnp.where(lenr == 1.0, 0.5, f_loc.astype(jnp.float32) / den)
    pos = alpha * 100.0

    kidx = jax.lax.broadcasted_iota(jnp.int32, (1, DM), 1)
    div = jnp.exp((2 * (kidx // 2)).astype(jnp.float32)
                  * (-math.log(10000.0) / DM))
    ang = pos * div
    pe = jnp.where(kidx % 2 == 0, jnp.sin(ang), jnp.cos(ang))

    x = _bdot(_gelu(_bdot(pe, ap1w_ref[...]) + ap1b_ref[...]),
              ap2w_ref[...]) + ap2b_ref[...]

    cmask = (b_loc == jax.lax.broadcasted_iota(
        jnp.int32, (1, R2), 1) // F_MAX)

    nt_dims = (((1,), (1,)), ((), ()))

    for (l1g, l1b, sawq, sabq, sawk, sabk, sawv, sabv, sawo, sabo,
         l2g, l2b, cawq, cabq, cawk, cabk, cawv, cabv, cawo, cabo,
         l3g, l3b, ff1w, ff1b, ff2w, ff2b) in lyr:
        h = _ln(x, l1g[...], l1b[...])
        q = _bdot(h, sawq[...]) + sabq[...]
        k = _bdot(h, sawk[...]) + sabk[...]
        v = _bdot(h, sawv[...]) + sabv[...]
        heads = []
        for hh in range(NHEAD):
            sl = slice(hh * DH, (hh + 1) * DH)
            s = jax.lax.dot_general(q[:, sl].astype(_BF), k[:, sl].astype(_BF),
                                    nt_dims, preferred_element_type=jnp.float32)
            s = jnp.where(cmask, s * (1.0 / math.sqrt(DH)), -1e9)
            s = s - jnp.max(s, axis=1, keepdims=True)
            es = jnp.exp(s)
            a = es / jnp.sum(es, axis=1, keepdims=True)
            heads.append(_bdot(a, v[:, sl]))
        o = jnp.concatenate(heads, axis=1)
        x = x + _bdot(o, sawo[...]) + sabo[...]
        h = _ln(x, l2g[...], l2b[...])
        q = _bdot(h, cawq[...]) + cabq[...]
        kc = _bdot(mem_cat, cawk[...]) + cabk[...]
        vc = _bdot(mem_cat, cawv[...]) + cabv[...]
        kt = [_bdot(E, kc[t * BK2:(t + 1) * BK2, :]) for t in range(3)]
        vt = [_bdot(E, vc[t * BK2:(t + 1) * BK2, :]) for t in range(3)]
        heads = []
        for hh in range(NHEAD):
            sl = slice(hh * DH, (hh + 1) * DH)
            qh = q[:, sl].astype(_BF).astype(jnp.float32)
            st = [jnp.sum(qh * kt[t][:, sl].astype(_BF).astype(jnp.float32),
                          axis=1, keepdims=True) * (1.0 / math.sqrt(DH))
                  for t in range(3)]
            m = jnp.maximum(jnp.maximum(st[0], st[1]), st[2])
            et = [jnp.exp(s - m) for s in st]
            zden = et[0] + et[1] + et[2]
            oh = (et[0] * vt[0][:, sl] + et[1] * vt[1][:, sl]
                  + et[2] * vt[2][:, sl]) / zden
            heads.append(oh)
        o = jnp.concatenate(heads, axis=1)
        x = x + _bdot(o, cawo[...]) + cabo[...]
        h = _ln(x, l3g[...], l3b[...])
        ffa = jnp.maximum(_bdot(h, ff1w[...]) + ff1b[...], 0.0)
        x = x + _bdot(ffa, ff2w[...]) + ff2b[...]

    recon_o[...] = _bdot(x, outw_ref[...]) + outb_ref[...]


def kernel(frames, ph_ids, lengths, params):
    b = frames.shape[0]
    nb1 = b // BK1
    nb2 = b // BK2

    f2 = jnp.pad(frames, ((0, 0), (0, FPAD - F_MAX), (0, 0))
                 ).reshape(b * FPAD, IN_DIM)
    phid = ph_ids.reshape(b, 1).astype(jnp.int32)
    lens2 = lengths.reshape(b, 1).astype(jnp.int32)

    def row2(a):
        return a.reshape(1, -1) if a.ndim == 1 else a

    def enc_list(p):
        out = []
        for kk in _ENC_KEYS:
            a = p[kk]
            if kk == 'attn_q':
                a = jnp.tile(a.reshape(-1, 1), (1, 8))
            out.append(row2(a))
        return out

    enc_s = enc_list(params['enc_start'])
    enc_e = enc_list(params['enc_end'])
    cb_s = params['cb_start'].reshape(NQ * CB, LAT)
    cb_e = params['cb_end'].reshape(NQ * CB, LAT)

    consts1 = ([params['ph_emb']] + enc_s + enc_e +
               [cb_s, cb_s.T, cb_e, cb_e.T,
                params['p2d_w'], row2(params['p2d_b']),
                params['mem_w'], row2(params['mem_b']), params['mem_type']])

    def cspec(a):
        nd = a.ndim
        return pl.BlockSpec(a.shape, lambda i, _nd=nd: (0,) * _nd)

    out1 = pl.pallas_call(
        _enc_rvq_body,
        grid=(nb1,),
        in_specs=[
            pl.BlockSpec((BK1 * FPAD, IN_DIM), lambda i: (i, 0)),
            pl.BlockSpec((BK1, 1), lambda i: (i, 0)),
            pl.BlockSpec((BK1, 1), lambda i: (i, 0)),
        ] + [cspec(a) for a in consts1],
        out_specs=[
            pl.BlockSpec((BK1, DM), lambda i: (i, 0)),
            pl.BlockSpec((BK1, DM), lambda i: (i, 0)),
            pl.BlockSpec((BK1, DM), lambda i: (i, 0)),
            pl.BlockSpec((BK1, NQ), lambda i: (i, 0)),
            pl.BlockSpec((BK1, NQ), lambda i: (i, 0)),
            pl.BlockSpec((1, 1), lambda i: (0, 0)),
        ],
        out_shape=[
            jax.ShapeDtypeStruct((b, DM), jnp.float32),
            jax.ShapeDtypeStruct((b, DM), jnp.float32),
            jax.ShapeDtypeStruct((b, DM), jnp.float32),
            jax.ShapeDtypeStruct((b, NQ), jnp.int32),
            jax.ShapeDtypeStruct((b, NQ), jnp.int32),
            jax.ShapeDtypeStruct((1, 1), jnp.float32),
        ],
    )(f2, phid, lens2, *consts1)
    mem_s, mem_e, mem_p, s_idx, e_idx, commit11 = out1

    mem3 = jnp.stack([mem_s, mem_e, mem_p], axis=1)
    memblk = mem3.reshape(nb2, BK2, 3, DM).transpose(0, 2, 1, 3
                                                    ).reshape(nb2, 3 * BK2, DM)
    len3 = lengths.reshape(nb2, 1, BK2).astype(jnp.int32)

    lyr_consts = []
    for lp in params['layers']:
        lyr_consts += [row2(lp[kk]) for kk in _LAYER_KEYS]
    consts2 = ([params['ap1_w'], row2(params['ap1_b']),
                params['ap2_w'], row2(params['ap2_b'])] + lyr_consts +
               [params['outp_w'], row2(params['outp_b'])])

    recon2 = pl.pallas_call(
        _dec_body,
        grid=(nb2,),
        in_specs=[
            pl.BlockSpec((1, 3 * BK2, DM), lambda i: (i, 0, 0)),
            pl.BlockSpec((1, 1, BK2), lambda i: (i, 0, 0)),
        ] + [cspec(a) for a in consts2],
        out_specs=pl.BlockSpec((R2, IN_DIM), lambda i: (i, 0)),
        out_shape=jax.ShapeDtypeStruct((b * F_MAX, IN_DIM), jnp.float32),
    )(memblk, len3, *consts2)

    recon = recon2.reshape(b, F_MAX, IN_DIM)
    commit = (commit11 * (COMMIT_W / (b * LAT))).reshape(())
    return recon, s_idx, e_idx, commit
